# baseline (speedup 1.0000x reference)
import functools

import jax
import jax.numpy as jnp
from jax import lax
from jax.experimental import pallas as pl
from jax.experimental.pallas import tpu as pltpu
from jax.experimental.pallas import tpu_sc as plsc

B, S, D = 4, 4096, 1024
A = 1024
H = 16
K = 1024

_f32 = jnp.float32
_CONTRACT_LAST = (((1,), (1,)), ((), ()))


_bf16 = jnp.bfloat16


def _r16(x):
    return x.astype(_bf16).astype(_f32)


def _score_body(w_ref, v_ref, mem_ref, o_ref, u0_s):
    @pl.when(jnp.logical_and(pl.program_id(0) == 0, pl.program_id(1) == 0))
    def _():
        u0_s[...] = lax.dot_general(
            _r16(w_ref[...]), v_ref[...], _CONTRACT_LAST,
            preferred_element_type=_f32,
            precision=lax.Precision.HIGHEST)

    o_ref[0, 0] = lax.dot_general(
        _r16(mem_ref[0]), u0_s[...], (((1,), (0,)), ((), ())),
        preferred_element_type=_f32,
        precision=lax.Precision.HIGHEST)


_S0_NC = 4


def _score_call(w, v2, memory):
    nc = _S0_NC
    return pl.pallas_call(
        _score_body,
        grid=(B, nc),
        in_specs=[
            pl.BlockSpec((D, A), lambda b, c: (0, 0)),
            pl.BlockSpec((1, A), lambda b, c: (0, 0)),
            pl.BlockSpec((1, S // nc, D), lambda b, c: (b, c, 0)),
        ],
        out_specs=pl.BlockSpec((1, 1, S // nc, 1), lambda b, c: (b, c, 0, 0)),
        out_shape=jax.ShapeDtypeStruct((B, nc, S // nc, 1), _f32),
        scratch_shapes=[pltpu.VMEM((D, 1), _f32)],
    )(w, v2, memory)


def _u_heads_body(ws_ref, vs_ref, o_ref):
    o_ref[0] = lax.dot_general(
        _r16(ws_ref[0]), vs_ref[0], _CONTRACT_LAST,
        preferred_element_type=_f32,
        precision=lax.Precision.HIGHEST)


def _u_heads_call(ws, vs3):
    return pl.pallas_call(
        _u_heads_body,
        grid=(H,),
        in_specs=[
            pl.BlockSpec((1, A, A), lambda h: (h, 0, 0)),
            pl.BlockSpec((1, 1, A), lambda h: (h, 0, 0)),
        ],
        out_specs=pl.BlockSpec((1, A, 1), lambda h: (h, 0, 0)),
        out_shape=jax.ShapeDtypeStruct((H, A, 1), _f32),
    )(ws, vs3)


_RCH = 256
_NO_TIE_TOTAL = float(S * (S - 1) // 2)


def _topk_body(srow_ref, scol_ref, o_ref, rk_s):
    row = srow_ref[0]

    def fast(c, acc):
        scj = scol_ref[0, pl.ds(c * _RCH, _RCH), :]
        gt = scj > row
        return acc + jnp.sum(jnp.where(gt, 1.0, 0.0), axis=0, keepdims=True)

    ranks = lax.fori_loop(0, S // _RCH, fast, jnp.zeros((1, S), _f32))
    rk_s[...] = ranks
    total = jnp.sum(ranks)

    @pl.when(total != _NO_TIE_TOTAL)
    def _with_ties():
        ii = lax.broadcasted_iota(jnp.int32, (1, S), 1)

        def slow(c, acc):
            scj = scol_ref[0, pl.ds(c * _RCH, _RCH), :]
            jj = c * _RCH + lax.broadcasted_iota(jnp.int32, (_RCH, 1), 0)
            hit = (scj > row) | ((scj == row) & (jj < ii))
            return acc + jnp.sum(jnp.where(hit, 1.0, 0.0),
                                 axis=0, keepdims=True)

        rk_s[...] = lax.fori_loop(0, S // _RCH, slow, jnp.zeros((1, S), _f32))

    ranks = rk_s[...]
    ii_f = lax.broadcasted_iota(jnp.int32, (S, 1), 0).astype(_f32)
    cols = []
    for rc in range(K // _RCH):
        r_col = (rc * _RCH + lax.broadcasted_iota(
            jnp.int32, (_RCH, 1), 0)).astype(_f32)
        match = jnp.where(ranks == r_col, 1.0, 0.0)
        cols.append(lax.dot_general(
            match, ii_f, (((1,), (0,)), ((), ())),
            preferred_element_type=_f32,
            precision=lax.Precision.HIGHEST))
    base = (pl.program_id(0) * S).astype(_f32)
    o_ref[0] = (jnp.concatenate(cols, axis=0) + base).astype(jnp.int32)


def _topk_call(s_row, s_col):
    return pl.pallas_call(
        _topk_body,
        grid=(B,),
        in_specs=[
            pl.BlockSpec((1, 1, S), lambda b: (b, 0, 0)),
            pl.BlockSpec((1, S, 1), lambda b: (b, 0, 0)),
        ],
        out_specs=pl.BlockSpec((1, K, 1), lambda b: (b, 0, 0)),
        out_shape=jax.ShapeDtypeStruct((B, K, 1), jnp.int32),
        scratch_shapes=[pltpu.VMEM((1, S), _f32)],
    )(s_row, s_col)


_SC_NC, _SC_NS = 2, 16
_SC_NW = _SC_NC * _SC_NS
_ROWS_PER_W = (B * K) // _SC_NW
_GCHUNK = 64


def _sc_gather(mem_flat, gidx):
    mesh = plsc.VectorSubcoreMesh(
        core_axis_name="c", subcore_axis_name="s",
        num_cores=_SC_NC, num_subcores=_SC_NS)

    @functools.partial(
        pl.kernel,
        mesh=mesh,
        out_type=jax.ShapeDtypeStruct((B * K, D), _f32),
        scratch_types=[
            pltpu.VMEM((_GCHUNK,), jnp.int32),
            pltpu.VMEM((_GCHUNK, D), _f32),
            pltpu.SemaphoreType.DMA,
        ],
    )
    def gather_kernel(mem_hbm, idx_hbm, out_hbm, idx_v, rows_v, sem):
        wid = lax.axis_index("s") * _SC_NC + lax.axis_index("c")
        base = wid * _ROWS_PER_W
        for ch in range(_ROWS_PER_W // _GCHUNK):
            off = base + ch * _GCHUNK
            pltpu.sync_copy(idx_hbm.at[pl.ds(off, _GCHUNK)], idx_v)
            pltpu.async_copy(mem_hbm.at[idx_v], rows_v, sem).wait()
            pltpu.sync_copy(rows_v, out_hbm.at[pl.ds(off, _GCHUNK)])

    return gather_kernel(mem_flat, gidx)


def _attn_body(gv_ref, w_ref, b_ref, v_ref, u_ref, attn_ref, prob_ref):
    g = gv_ref[0].astype(_bf16)
    wb = w_ref[...].astype(_bf16)
    val = (lax.dot_general(g, wb, (((1,), (0,)), ((), ())),
                           preferred_element_type=_f32)
           + b_ref[...]) * v_ref[...]
    s1 = lax.dot_general(u_ref[...], _r16(val), _CONTRACT_LAST,
                         preferred_element_type=_f32,
                         precision=lax.Precision.HIGHEST)
    m = jnp.max(s1, axis=1, keepdims=True)
    e = jnp.exp(s1 - m)
    z = jnp.sum(e, axis=1, keepdims=True)
    p = e / z
    prob_ref[0] = p
    attn_ref[0] = lax.dot_general(p, val, (((1,), (0,)), ((), ())),
                                  preferred_element_type=_f32,
                                  precision=lax.Precision.HIGHEST)


def _attn_call(gv3, w, b2, v2, U2):
    return pl.pallas_call(
        _attn_body,
        grid=(B,),
        in_specs=[
            pl.BlockSpec((1, K, D), lambda b: (b, 0, 0)),
            pl.BlockSpec((D, A), lambda b: (0, 0)),
            pl.BlockSpec((1, A), lambda b: (0, 0)),
            pl.BlockSpec((1, A), lambda b: (0, 0)),
            pl.BlockSpec((H, A), lambda b: (0, 0)),
        ],
        out_specs=[
            pl.BlockSpec((1, H, A), lambda b: (b, 0, 0)),
            pl.BlockSpec((1, H, K), lambda b: (b, 0, 0)),
        ],
        out_shape=[
            jax.ShapeDtypeStruct((B, H, A), _f32),
            jax.ShapeDtypeStruct((B, H, K), _f32),
        ],
    )(gv3, w, b2, v2, U2)


def kernel(memory, w, b, v, ws, bs, vs):
    del bs
    v2 = v.reshape(1, A)
    vs3 = vs.reshape(H, 1, A)
    b2 = b.reshape(1, A)

    s0 = _score_call(w, v2, memory).reshape(B, S)
    gidx = _topk_call(s0.reshape(B, 1, S), s0.reshape(B, S, 1))
    gv = _sc_gather(memory.reshape(B * S, D), gidx.reshape(B * K))
    U2 = _u_heads_call(ws, vs3).reshape(H, A)
    attn, prob = _attn_call(gv.reshape(B, K, D), w, b2, v2, U2)
    return attn, prob


# speedup vs baseline: 1.2155x; 1.2155x over previous
import functools

import jax
import jax.numpy as jnp
from jax import lax
from jax.experimental import pallas as pl
from jax.experimental.pallas import tpu as pltpu
from jax.experimental.pallas import tpu_sc as plsc

B, S, D = 4, 4096, 1024
A = 1024
H = 16
K = 1024

_f32 = jnp.float32
_CONTRACT_LAST = (((1,), (1,)), ((), ()))


_bf16 = jnp.bfloat16


def _r16(x):
    return x.astype(_bf16).astype(_f32)


def _score_body(w_ref, v_ref, mem_ref, o_ref, u0_s):
    @pl.when(jnp.logical_and(pl.program_id(0) == 0, pl.program_id(1) == 0))
    def _():
        u0_s[...] = lax.dot_general(
            _r16(w_ref[...]), v_ref[...], _CONTRACT_LAST,
            preferred_element_type=_f32,
            precision=lax.Precision.HIGHEST)

    o_ref[0, 0] = lax.dot_general(
        _r16(mem_ref[0]), u0_s[...], (((1,), (0,)), ((), ())),
        preferred_element_type=_f32,
        precision=lax.Precision.HIGHEST)


_S0_NC = 4


def _score_call(w, v2, memory):
    nc = _S0_NC
    return pl.pallas_call(
        _score_body,
        grid=(B, nc),
        in_specs=[
            pl.BlockSpec((D, A), lambda b, c: (0, 0)),
            pl.BlockSpec((1, A), lambda b, c: (0, 0)),
            pl.BlockSpec((1, S // nc, D), lambda b, c: (b, c, 0)),
        ],
        out_specs=pl.BlockSpec((1, 1, S // nc, 1), lambda b, c: (b, c, 0, 0)),
        out_shape=jax.ShapeDtypeStruct((B, nc, S // nc, 1), _f32),
        scratch_shapes=[pltpu.VMEM((D, 1), _f32)],
    )(w, v2, memory)


def _u_heads_body(ws_ref, vs_ref, o_ref):
    o_ref[0] = lax.dot_general(
        _r16(ws_ref[0]), vs_ref[0], _CONTRACT_LAST,
        preferred_element_type=_f32,
        precision=lax.Precision.HIGHEST)


def _u_heads_call(ws, vs3):
    return pl.pallas_call(
        _u_heads_body,
        grid=(H,),
        in_specs=[
            pl.BlockSpec((1, A, A), lambda h: (h, 0, 0)),
            pl.BlockSpec((1, 1, A), lambda h: (h, 0, 0)),
        ],
        out_specs=pl.BlockSpec((1, A, 1), lambda h: (h, 0, 0)),
        out_shape=jax.ShapeDtypeStruct((H, A, 1), _f32),
    )(ws, vs3)


_TOPK_CH = 256
_NO_TIE_TOTAL = float(S * (S - 1) // 2)


def _topk_body(srow_ref, scol_ref, o_ref, rk_s):
    row = srow_ref[0]
    nch = S // _TOPK_CH

    def fast(c, _):
        sc = scol_ref[0, pl.ds(c * _TOPK_CH, _TOPK_CH), :]
        gt = row > sc
        rk_s[pl.ds(c * _TOPK_CH, _TOPK_CH), :] = jnp.sum(
            jnp.where(gt, 1.0, 0.0), axis=1, keepdims=True)
        return 0

    lax.fori_loop(0, nch, fast, 0)
    total = jnp.sum(rk_s[...])

    @pl.when(total != _NO_TIE_TOTAL)
    def _with_ties():
        jj = lax.broadcasted_iota(jnp.int32, (1, S), 1)

        def slow(c, _):
            sc = scol_ref[0, pl.ds(c * _TOPK_CH, _TOPK_CH), :]
            ii = c * _TOPK_CH + lax.broadcasted_iota(
                jnp.int32, (_TOPK_CH, 1), 0)
            hit = (row > sc) | ((row == sc) & (jj < ii))
            rk_s[pl.ds(c * _TOPK_CH, _TOPK_CH), :] = jnp.sum(
                jnp.where(hit, 1.0, 0.0), axis=1, keepdims=True)
            return 0

        lax.fori_loop(0, nch, slow, 0)

    rr = lax.broadcasted_iota(jnp.int32, (1, K), 1).astype(_f32)

    def slot(c, acc):
        rk = rk_s[pl.ds(c * _TOPK_CH, _TOPK_CH), :]
        ii = (c * _TOPK_CH + lax.broadcasted_iota(
            jnp.int32, (_TOPK_CH, 1), 0)).astype(_f32)
        contrib = jnp.sum(jnp.where(rk == rr, ii, 0.0),
                          axis=0, keepdims=True)
        return acc + contrib

    acc = lax.fori_loop(0, nch, slot, jnp.zeros((1, K), _f32))
    base = (pl.program_id(0) * S).astype(_f32)
    o_ref[0] = (acc + base).astype(jnp.int32)


def _topk_call(s_row, s_col):
    return pl.pallas_call(
        _topk_body,
        grid=(B,),
        in_specs=[
            pl.BlockSpec((1, 1, S), lambda b: (b, 0, 0)),
            pl.BlockSpec((1, S, 1), lambda b: (b, 0, 0)),
        ],
        out_specs=pl.BlockSpec((1, 1, K), lambda b: (b, 0, 0)),
        out_shape=jax.ShapeDtypeStruct((B, 1, K), jnp.int32),
        scratch_shapes=[pltpu.VMEM((S, 1), _f32)],
    )(s_row, s_col)


_SC_NC, _SC_NS = 2, 16
_SC_NW = _SC_NC * _SC_NS
_ROWS_PER_W = (B * K) // _SC_NW
_GCHUNK = 64


def _sc_gather(mem_flat, gidx):
    mesh = plsc.VectorSubcoreMesh(
        core_axis_name="c", subcore_axis_name="s",
        num_cores=_SC_NC, num_subcores=_SC_NS)

    @functools.partial(
        pl.kernel,
        mesh=mesh,
        out_type=jax.ShapeDtypeStruct((B * K, D), _f32),
        scratch_types=[
            pltpu.VMEM((_GCHUNK,), jnp.int32),
            pltpu.VMEM((_GCHUNK, D), _f32),
            pltpu.SemaphoreType.DMA,
        ],
    )
    def gather_kernel(mem_hbm, idx_hbm, out_hbm, idx_v, rows_v, sem):
        wid = lax.axis_index("s") * _SC_NC + lax.axis_index("c")
        base = wid * _ROWS_PER_W
        for ch in range(_ROWS_PER_W // _GCHUNK):
            off = base + ch * _GCHUNK
            pltpu.sync_copy(idx_hbm.at[pl.ds(off, _GCHUNK)], idx_v)
            pltpu.async_copy(mem_hbm.at[idx_v], rows_v, sem).wait()
            pltpu.sync_copy(rows_v, out_hbm.at[pl.ds(off, _GCHUNK)])

    return gather_kernel(mem_flat, gidx)


def _attn_body(gv_ref, w_ref, b_ref, v_ref, u_ref, attn_ref, prob_ref):
    g = gv_ref[0].astype(_bf16)
    wb = w_ref[...].astype(_bf16)
    val = (lax.dot_general(g, wb, (((1,), (0,)), ((), ())),
                           preferred_element_type=_f32)
           + b_ref[...]) * v_ref[...]
    s1 = lax.dot_general(u_ref[...], _r16(val), _CONTRACT_LAST,
                         preferred_element_type=_f32,
                         precision=lax.Precision.HIGHEST)
    m = jnp.max(s1, axis=1, keepdims=True)
    e = jnp.exp(s1 - m)
    z = jnp.sum(e, axis=1, keepdims=True)
    p = e / z
    prob_ref[0] = p
    attn_ref[0] = lax.dot_general(p, val, (((1,), (0,)), ((), ())),
                                  preferred_element_type=_f32,
                                  precision=lax.Precision.HIGHEST)


def _attn_call(gv3, w, b2, v2, U2):
    return pl.pallas_call(
        _attn_body,
        grid=(B,),
        in_specs=[
            pl.BlockSpec((1, K, D), lambda b: (b, 0, 0)),
            pl.BlockSpec((D, A), lambda b: (0, 0)),
            pl.BlockSpec((1, A), lambda b: (0, 0)),
            pl.BlockSpec((1, A), lambda b: (0, 0)),
            pl.BlockSpec((H, A), lambda b: (0, 0)),
        ],
        out_specs=[
            pl.BlockSpec((1, H, A), lambda b: (b, 0, 0)),
            pl.BlockSpec((1, H, K), lambda b: (b, 0, 0)),
        ],
        out_shape=[
            jax.ShapeDtypeStruct((B, H, A), _f32),
            jax.ShapeDtypeStruct((B, H, K), _f32),
        ],
    )(gv3, w, b2, v2, U2)


def kernel(memory, w, b, v, ws, bs, vs):
    del bs
    v2 = v.reshape(1, A)
    vs3 = vs.reshape(H, 1, A)
    b2 = b.reshape(1, A)

    s0 = _score_call(w, v2, memory).reshape(B, S)
    gidx = _topk_call(s0.reshape(B, 1, S), s0.reshape(B, S, 1))
    gv = _sc_gather(memory.reshape(B * S, D), gidx.reshape(B * K))
    U2 = _u_heads_call(ws, vs3).reshape(H, A)
    attn, prob = _attn_call(gv.reshape(B, K, D), w, b2, v2, U2)
    return attn, prob


# speedup vs baseline: 1.5751x; 1.2959x over previous
import functools

import jax
import jax.numpy as jnp
from jax import lax
from jax.experimental import pallas as pl
from jax.experimental.pallas import tpu as pltpu
from jax.experimental.pallas import tpu_sc as plsc

B, S, D = 4, 4096, 1024
A = 1024
H = 16
K = 1024

_f32 = jnp.float32
_CONTRACT_LAST = (((1,), (1,)), ((), ()))


_bf16 = jnp.bfloat16


def _r16(x):
    return x.astype(_bf16).astype(_f32)


def _matvec_body(w_ref, v_ref, o_ref):
    o_ref[...] = lax.dot_general(
        _r16(w_ref[...]), v_ref[...], _CONTRACT_LAST,
        preferred_element_type=_f32,
        precision=lax.Precision.HIGHEST)


def _u0_call(w, v2):
    return pl.pallas_call(
        _matvec_body,
        out_shape=jax.ShapeDtypeStruct((D, 1), _f32),
    )(w, v2)


def _s0_body(mem_ref, u_ref, o_ref):
    o_ref[0, 0] = lax.dot_general(
        _r16(mem_ref[0]), u_ref[...], _CONTRACT_LAST,
        preferred_element_type=_f32,
        precision=lax.Precision.HIGHEST)


def _s0_call(memory, u0r):
    nc = 4
    return pl.pallas_call(
        _s0_body,
        grid=(B, nc),
        in_specs=[
            pl.BlockSpec((1, S // nc, D), lambda b, c: (b, c, 0)),
            pl.BlockSpec((1, D), lambda b, c: (0, 0)),
        ],
        out_specs=pl.BlockSpec((1, 1, S // nc, 1), lambda b, c: (b, c, 0, 0)),
        out_shape=jax.ShapeDtypeStruct((B, nc, S // nc, 1), _f32),
    )(memory, u0r)


def _u_heads_body(ws_ref, vs_ref, o_ref):
    o_ref[0] = lax.dot_general(
        _r16(ws_ref[0]), vs_ref[0], _CONTRACT_LAST,
        preferred_element_type=_f32,
        precision=lax.Precision.HIGHEST)


def _u_heads_call(ws, vs3):
    return pl.pallas_call(
        _u_heads_body,
        grid=(H,),
        in_specs=[
            pl.BlockSpec((1, A, A), lambda h: (h, 0, 0)),
            pl.BlockSpec((1, 1, A), lambda h: (h, 0, 0)),
        ],
        out_specs=pl.BlockSpec((1, A, 1), lambda h: (h, 0, 0)),
        out_shape=jax.ShapeDtypeStruct((H, A, 1), _f32),
    )(ws, vs3)


_TOPK_CH = 256


def _topk_body(srow_ref, scol_ref, o_ref):
    row = srow_ref[0]
    jj = lax.broadcasted_iota(jnp.int32, (1, S), 1)
    rr = lax.broadcasted_iota(jnp.int32, (1, K), 1).astype(_f32)

    def body(c, acc):
        sc = scol_ref[0, pl.ds(c * _TOPK_CH, _TOPK_CH), :]
        ii = c * _TOPK_CH + lax.broadcasted_iota(
            jnp.int32, (_TOPK_CH, 1), 0)
        gt = row > sc
        eq = (row == sc) & (jj < ii)
        cnt = jnp.sum(jnp.where(gt | eq, 1.0, 0.0),
                      axis=1, keepdims=True)
        match = cnt == rr
        contrib = jnp.sum(jnp.where(match, ii.astype(_f32), 0.0),
                          axis=0, keepdims=True)
        return acc + contrib

    acc = lax.fori_loop(0, S // _TOPK_CH, body, jnp.zeros((1, K), _f32))
    base = (pl.program_id(0) * S).astype(_f32)
    o_ref[0] = (acc + base).astype(jnp.int32)


def _topk_call(s_row, s_col):
    return pl.pallas_call(
        _topk_body,
        grid=(B,),
        in_specs=[
            pl.BlockSpec((1, 1, S), lambda b: (b, 0, 0)),
            pl.BlockSpec((1, S, 1), lambda b: (b, 0, 0)),
        ],
        out_specs=pl.BlockSpec((1, 1, K), lambda b: (b, 0, 0)),
        out_shape=jax.ShapeDtypeStruct((B, 1, K), jnp.int32),
    )(s_row, s_col)


_SC_NC, _SC_NS = 2, 16
_SC_NW = _SC_NC * _SC_NS
_ROWS_PER_W = (B * K) // _SC_NW
_GCHUNK = 64


def _sc_gather(mem_flat, gidx):
    mesh = plsc.VectorSubcoreMesh(
        core_axis_name="c", subcore_axis_name="s",
        num_cores=_SC_NC, num_subcores=_SC_NS)

    @functools.partial(
        pl.kernel,
        mesh=mesh,
        out_type=jax.ShapeDtypeStruct((B * K, D), _f32),
        scratch_types=[
            pltpu.VMEM((_GCHUNK,), jnp.int32),
            pltpu.VMEM((_GCHUNK, D), _f32),
            pltpu.SemaphoreType.DMA,
        ],
    )
    def gather_kernel(mem_hbm, idx_hbm, out_hbm, idx_v, rows_v, sem):
        wid = lax.axis_index("s") * _SC_NC + lax.axis_index("c")
        base = wid * _ROWS_PER_W
        for ch in range(_ROWS_PER_W // _GCHUNK):
            off = base + ch * _GCHUNK
            pltpu.sync_copy(idx_hbm.at[pl.ds(off, _GCHUNK)], idx_v)
            pltpu.async_copy(mem_hbm.at[idx_v], rows_v, sem).wait()
            pltpu.sync_copy(rows_v, out_hbm.at[pl.ds(off, _GCHUNK)])

    return gather_kernel(mem_flat, gidx)


def _attn_body(gv_ref, w_ref, b_ref, v_ref, u_ref, attn_ref, prob_ref):
    g = gv_ref[0].astype(_bf16)
    wb = w_ref[...].astype(_bf16)
    val = (lax.dot_general(g, wb, (((1,), (0,)), ((), ())),
                           preferred_element_type=_f32)
           + b_ref[...]) * v_ref[...]
    s1 = lax.dot_general(u_ref[...], _r16(val), _CONTRACT_LAST,
                         preferred_element_type=_f32,
                         precision=lax.Precision.HIGHEST)
    m = jnp.max(s1, axis=1, keepdims=True)
    e = jnp.exp(s1 - m)
    z = jnp.sum(e, axis=1, keepdims=True)
    p = e / z
    prob_ref[0] = p
    attn_ref[0] = lax.dot_general(p, val, (((1,), (0,)), ((), ())),
                                  preferred_element_type=_f32,
                                  precision=lax.Precision.HIGHEST)


def _attn_call(gv3, w, b2, v2, U2):
    return pl.pallas_call(
        _attn_body,
        grid=(B,),
        in_specs=[
            pl.BlockSpec((1, K, D), lambda b: (b, 0, 0)),
            pl.BlockSpec((D, A), lambda b: (0, 0)),
            pl.BlockSpec((1, A), lambda b: (0, 0)),
            pl.BlockSpec((1, A), lambda b: (0, 0)),
            pl.BlockSpec((H, A), lambda b: (0, 0)),
        ],
        out_specs=[
            pl.BlockSpec((1, H, A), lambda b: (b, 0, 0)),
            pl.BlockSpec((1, H, K), lambda b: (b, 0, 0)),
        ],
        out_shape=[
            jax.ShapeDtypeStruct((B, H, A), _f32),
            jax.ShapeDtypeStruct((B, H, K), _f32),
        ],
    )(gv3, w, b2, v2, U2)


def kernel(memory, w, b, v, ws, bs, vs):
    del bs
    v2 = v.reshape(1, A)
    vs3 = vs.reshape(H, 1, A)
    b2 = b.reshape(1, A)

    u0 = _u0_call(w, v2).reshape(1, D)
    s0 = _s0_call(memory, u0).reshape(B, S)
    gidx = _topk_call(s0.reshape(B, 1, S), s0.reshape(B, S, 1))
    gv = _sc_gather(memory.reshape(B * S, D), gidx.reshape(B * K))
    U2 = _u_heads_call(ws, vs3).reshape(H, A)
    attn, prob = _attn_call(gv.reshape(B, K, D), w, b2, v2, U2)
    return attn, prob


# speedup vs baseline: 1.5884x; 1.0084x over previous
import functools

import jax
import jax.numpy as jnp
from jax import lax
from jax.experimental import pallas as pl
from jax.experimental.pallas import tpu as pltpu
from jax.experimental.pallas import tpu_sc as plsc

B, S, D = 4, 4096, 1024
A = 1024
H = 16
K = 1024

_f32 = jnp.float32
_CONTRACT_LAST = (((1,), (1,)), ((), ()))


_bf16 = jnp.bfloat16


def _r16(x):
    return x.astype(_bf16).astype(_f32)


def _matvec_body(w_ref, v_ref, o_ref):
    o_ref[...] = lax.dot_general(
        _r16(w_ref[...]), v_ref[...], _CONTRACT_LAST,
        preferred_element_type=_f32,
        precision=lax.Precision.HIGHEST)


def _u0_call(w, v2):
    return pl.pallas_call(
        _matvec_body,
        out_shape=jax.ShapeDtypeStruct((D, 1), _f32),
    )(w, v2)


def _s0_body(mem_ref, u_ref, o_ref):
    o_ref[0, 0] = lax.dot_general(
        _r16(mem_ref[0]), u_ref[...], _CONTRACT_LAST,
        preferred_element_type=_f32,
        precision=lax.Precision.HIGHEST)


def _s0_call(memory, u0r):
    nc = 4
    return pl.pallas_call(
        _s0_body,
        grid=(B, nc),
        in_specs=[
            pl.BlockSpec((1, S // nc, D), lambda b, c: (b, c, 0)),
            pl.BlockSpec((1, D), lambda b, c: (0, 0)),
        ],
        out_specs=pl.BlockSpec((1, 1, S // nc, 1), lambda b, c: (b, c, 0, 0)),
        out_shape=jax.ShapeDtypeStruct((B, nc, S // nc, 1), _f32),
    )(memory, u0r)


def _u_heads_body(ws_ref, vs_ref, o_ref):
    o_ref[0] = lax.dot_general(
        _r16(ws_ref[0]), vs_ref[0], _CONTRACT_LAST,
        preferred_element_type=_f32,
        precision=lax.Precision.HIGHEST)


def _u_heads_call(ws, vs3):
    return pl.pallas_call(
        _u_heads_body,
        grid=(H,),
        in_specs=[
            pl.BlockSpec((1, A, A), lambda h: (h, 0, 0)),
            pl.BlockSpec((1, 1, A), lambda h: (h, 0, 0)),
        ],
        out_specs=pl.BlockSpec((1, A, 1), lambda h: (h, 0, 0)),
        out_shape=jax.ShapeDtypeStruct((H, A, 1), _f32),
    )(ws, vs3)


_TOPK_CH = 512


def _topk_body(srow_ref, scol_ref, o_ref):
    row = srow_ref[0]
    jj = lax.broadcasted_iota(jnp.int32, (1, S), 1)
    rr = lax.broadcasted_iota(jnp.int32, (1, K), 1).astype(_f32)

    def body(c, acc):
        sc = scol_ref[0, pl.ds(c * _TOPK_CH, _TOPK_CH), :]
        ii = c * _TOPK_CH + lax.broadcasted_iota(
            jnp.int32, (_TOPK_CH, 1), 0)
        gt = row > sc
        eq = (row == sc) & (jj < ii)
        cnt = jnp.sum(jnp.where(gt | eq, 1.0, 0.0),
                      axis=1, keepdims=True)
        match = cnt == rr
        contrib = jnp.sum(jnp.where(match, ii.astype(_f32), 0.0),
                          axis=0, keepdims=True)
        return acc + contrib

    acc = lax.fori_loop(0, S // _TOPK_CH, body, jnp.zeros((1, K), _f32))
    base = (pl.program_id(0) * S).astype(_f32)
    o_ref[0] = (acc + base).astype(jnp.int32)


def _topk_call(s_row, s_col):
    return pl.pallas_call(
        _topk_body,
        grid=(B,),
        in_specs=[
            pl.BlockSpec((1, 1, S), lambda b: (b, 0, 0)),
            pl.BlockSpec((1, S, 1), lambda b: (b, 0, 0)),
        ],
        out_specs=pl.BlockSpec((1, 1, K), lambda b: (b, 0, 0)),
        out_shape=jax.ShapeDtypeStruct((B, 1, K), jnp.int32),
    )(s_row, s_col)


_SC_NC, _SC_NS = 2, 16
_SC_NW = _SC_NC * _SC_NS
_ROWS_PER_W = (B * K) // _SC_NW
_GCHUNK = 64


def _sc_gather(mem_flat, gidx):
    mesh = plsc.VectorSubcoreMesh(
        core_axis_name="c", subcore_axis_name="s",
        num_cores=_SC_NC, num_subcores=_SC_NS)

    @functools.partial(
        pl.kernel,
        mesh=mesh,
        out_type=jax.ShapeDtypeStruct((B * K, D), _f32),
        scratch_types=[
            pltpu.VMEM((_GCHUNK,), jnp.int32),
            pltpu.VMEM((_GCHUNK, D), _f32),
            pltpu.SemaphoreType.DMA,
        ],
    )
    def gather_kernel(mem_hbm, idx_hbm, out_hbm, idx_v, rows_v, sem):
        wid = lax.axis_index("s") * _SC_NC + lax.axis_index("c")
        base = wid * _ROWS_PER_W
        for ch in range(_ROWS_PER_W // _GCHUNK):
            off = base + ch * _GCHUNK
            pltpu.sync_copy(idx_hbm.at[pl.ds(off, _GCHUNK)], idx_v)
            pltpu.async_copy(mem_hbm.at[idx_v], rows_v, sem).wait()
            pltpu.sync_copy(rows_v, out_hbm.at[pl.ds(off, _GCHUNK)])

    return gather_kernel(mem_flat, gidx)


def _attn_body(gv_ref, w_ref, b_ref, v_ref, u_ref, attn_ref, prob_ref):
    g = gv_ref[0].astype(_bf16)
    wb = w_ref[...].astype(_bf16)
    val = (lax.dot_general(g, wb, (((1,), (0,)), ((), ())),
                           preferred_element_type=_f32)
           + b_ref[...]) * v_ref[...]
    s1 = lax.dot_general(u_ref[...], _r16(val), _CONTRACT_LAST,
                         preferred_element_type=_f32,
                         precision=lax.Precision.HIGHEST)
    m = jnp.max(s1, axis=1, keepdims=True)
    e = jnp.exp(s1 - m)
    z = jnp.sum(e, axis=1, keepdims=True)
    p = e / z
    prob_ref[0] = p
    attn_ref[0] = lax.dot_general(p, val, (((1,), (0,)), ((), ())),
                                  preferred_element_type=_f32,
                                  precision=lax.Precision.HIGHEST)


def _attn_call(gv3, w, b2, v2, U2):
    return pl.pallas_call(
        _attn_body,
        grid=(B,),
        in_specs=[
            pl.BlockSpec((1, K, D), lambda b: (b, 0, 0)),
            pl.BlockSpec((D, A), lambda b: (0, 0)),
            pl.BlockSpec((1, A), lambda b: (0, 0)),
            pl.BlockSpec((1, A), lambda b: (0, 0)),
            pl.BlockSpec((H, A), lambda b: (0, 0)),
        ],
        out_specs=[
            pl.BlockSpec((1, H, A), lambda b: (b, 0, 0)),
            pl.BlockSpec((1, H, K), lambda b: (b, 0, 0)),
        ],
        out_shape=[
            jax.ShapeDtypeStruct((B, H, A), _f32),
            jax.ShapeDtypeStruct((B, H, K), _f32),
        ],
    )(gv3, w, b2, v2, U2)


def kernel(memory, w, b, v, ws, bs, vs):
    del bs
    v2 = v.reshape(1, A)
    vs3 = vs.reshape(H, 1, A)
    b2 = b.reshape(1, A)

    u0 = _u0_call(w, v2).reshape(1, D)
    s0 = _s0_call(memory, u0).reshape(B, S)
    gidx = _topk_call(s0.reshape(B, 1, S), s0.reshape(B, S, 1))
    gv = _sc_gather(memory.reshape(B * S, D), gidx.reshape(B * K))
    U2 = _u_heads_call(ws, vs3).reshape(H, A)
    attn, prob = _attn_call(gv.reshape(B, K, D), w, b2, v2, U2)
    return attn, prob
